# baseline (speedup 1.0000x reference)
import functools

import jax
import jax.numpy as jnp
from jax import lax
from jax.experimental import pallas as pl
from jax.experimental.pallas import tpu as pltpu
from jax.experimental.pallas import tpu_sc as plsc

NTYPES = 95
DIM = 128
N_NODES = 100000

NW = 32
BPW = 3120
MAIN = NW * BPW
CH = 104
NCH = BPW // CH
NBUF = 5
LEAD = 3
NGRP = NCH // NBUF
TAIL = N_NODES - MAIN
TAIL_PER_W = 8
TAIL_WORKERS = TAIL // TAIL_PER_W

_mesh = plsc.VectorSubcoreMesh(core_axis_name="c", subcore_axis_name="s")


@functools.partial(
    pl.kernel,
    out_type=jax.ShapeDtypeStruct((N_NODES, DIM), jnp.float32),
    mesh=_mesh,
    scratch_types=[
        pltpu.VMEM((NCH, CH), jnp.int32),
        [pltpu.VMEM((CH, DIM), jnp.float32) for _ in range(NBUF)],
        pltpu.VMEM((TAIL_PER_W,), jnp.int32),
        pltpu.VMEM((TAIL_PER_W, DIM), jnp.float32),
        pltpu.VMEM((NTYPES, DIM), jnp.float32),
        pltpu.VMEM_SHARED((NTYPES, DIM), jnp.float32),
        [pltpu.SemaphoreType.DMA for _ in range(NBUF)],
        [pltpu.SemaphoreType.DMA for _ in range(NBUF)],
        pltpu.SemaphoreType.DMA,
    ],
)
def _emb_lookup(idx3d_hbm, tail_hbm, table_hbm, out_hbm,
                idx_v, rows, tidx_v, trows_v, table_l, table_sh,
                gsem, ssem, psem):
    wid = lax.axis_index("s") * 2 + lax.axis_index("c")
    base = pl.multiple_of(wid * BPW, 8)

    pltpu.async_copy(idx3d_hbm.at[wid], idx_v, psem)

    @pl.when(lax.axis_index("s") == 0)
    def _stage():
        pltpu.sync_copy(table_hbm, table_l)
        pltpu.sync_copy(table_l, table_sh)
    plsc.subcore_barrier()
    pltpu.make_async_copy(idx3d_hbm.at[wid], idx_v, psem).wait()

    def gather(c, b):
        pltpu.async_copy(table_sh.at[idx_v.at[c]], rows[b], gsem[b])

    def store(c, b):
        off = pl.multiple_of(base + c * CH, 8)
        pltpu.async_copy(rows[b], out_hbm.at[pl.ds(off, CH)], ssem[b])

    def wait_gather(b):
        pltpu.make_async_copy(table_sh.at[idx_v.at[0]], rows[b], gsem[b]).wait()

    def wait_store(b):
        pltpu.make_async_copy(rows[b], out_hbm.at[pl.ds(0, CH)], ssem[b]).wait()

    is_tail_worker = wid < TAIL_WORKERS

    @pl.when(is_tail_worker)
    def _tail_start():
        pltpu.sync_copy(tail_hbm.at[pl.ds(wid * TAIL_PER_W, TAIL_PER_W)], tidx_v)
        pltpu.async_copy(table_sh.at[tidx_v], trows_v, psem)

    for b in range(LEAD):
        gather(b, b)

    def body(g, carry):
        for b in range(NBUF):
            c = g * NBUF + b
            bg = (b + LEAD) % NBUF
            if b + LEAD - NBUF >= 0:
                wait_store(bg)
            else:
                @pl.when(g > 0)
                def _():
                    wait_store(bg)

            @pl.when(c + LEAD < NCH)
            def _():
                gather(c + LEAD, bg)

            wait_gather(b)
            store(c, b)
        return carry

    lax.fori_loop(0, NGRP, body, 0)

    @pl.when(is_tail_worker)
    def _tail_finish():
        toff = pl.multiple_of(MAIN + wid * TAIL_PER_W, 8)
        pltpu.make_async_copy(table_sh.at[tidx_v], trows_v, psem).wait()
        pltpu.sync_copy(trows_v, out_hbm.at[pl.ds(toff, TAIL_PER_W)])

    for k in range(NCH - (NBUF - LEAD), NCH):
        wait_store(k % NBUF)


def kernel(node_attr, embedding_table):
    idx = node_attr.astype(jnp.int32)
    idx_main = idx[:MAIN].reshape(NW, NCH, CH)
    idx_tail = idx[MAIN:]
    return _emb_lookup(idx_main, idx_tail, embedding_table)


# speedup vs baseline: 1.0202x; 1.0202x over previous
import functools

import jax
import jax.numpy as jnp
from jax import lax
from jax.experimental import pallas as pl
from jax.experimental.pallas import tpu as pltpu
from jax.experimental.pallas import tpu_sc as plsc

NTYPES = 95
DIM = 128
N_NODES = 100000

NW = 32
BPW = 3120
MAIN = NW * BPW
CH = 104
NCH = BPW // CH
NBUF = 5
LEAD = 2
NGRP = NCH // NBUF
STAGE_ROWS = 8
TAIL = N_NODES - MAIN
TAIL_PER_W = 8
TAIL_WORKERS = TAIL // TAIL_PER_W

_mesh = plsc.VectorSubcoreMesh(core_axis_name="c", subcore_axis_name="s")


@functools.partial(
    pl.kernel,
    out_type=jax.ShapeDtypeStruct((N_NODES, DIM), jnp.float32),
    mesh=_mesh,
    scratch_types=[
        pltpu.VMEM((NCH, CH), jnp.int32),
        [pltpu.VMEM((CH, DIM), jnp.float32) for _ in range(NBUF)],
        pltpu.VMEM((TAIL_PER_W,), jnp.int32),
        pltpu.VMEM((TAIL_PER_W, DIM), jnp.float32),
        pltpu.VMEM((STAGE_ROWS, DIM), jnp.float32),
        pltpu.VMEM_SHARED((NTYPES, DIM), jnp.float32),
        [pltpu.SemaphoreType.DMA for _ in range(NBUF)],
        [pltpu.SemaphoreType.DMA for _ in range(NBUF)],
        pltpu.SemaphoreType.DMA,
    ],
)
def _emb_lookup(idx3d_hbm, tail_hbm, table_hbm, out_hbm,
                idx_v, rows, tidx_v, trows_v, table_l, table_sh,
                gsem, ssem, psem):
    wid = lax.axis_index("s") * 2 + lax.axis_index("c")
    base = pl.multiple_of(wid * BPW, 8)

    pltpu.async_copy(idx3d_hbm.at[wid], idx_v, psem)

    sid = lax.axis_index("s")

    @pl.when(sid < NTYPES // STAGE_ROWS)
    def _stage_full():
        srow = pl.multiple_of(sid * STAGE_ROWS, 8)
        pltpu.sync_copy(table_hbm.at[pl.ds(srow, STAGE_ROWS)], table_l)
        pltpu.sync_copy(table_l, table_sh.at[pl.ds(srow, STAGE_ROWS)])

    @pl.when(sid == NTYPES // STAGE_ROWS)
    def _stage_rest():
        rest = NTYPES % STAGE_ROWS
        srow = (NTYPES // STAGE_ROWS) * STAGE_ROWS
        pltpu.sync_copy(table_hbm.at[pl.ds(srow, rest)], table_l.at[pl.ds(0, rest)])
        pltpu.sync_copy(table_l.at[pl.ds(0, rest)], table_sh.at[pl.ds(srow, rest)])

    plsc.subcore_barrier()
    pltpu.make_async_copy(idx3d_hbm.at[wid], idx_v, psem).wait()

    def gather(c, b):
        pltpu.async_copy(table_sh.at[idx_v.at[c]], rows[b], gsem[b])

    def store(c, b):
        off = pl.multiple_of(base + c * CH, 8)
        pltpu.async_copy(rows[b], out_hbm.at[pl.ds(off, CH)], ssem[b])

    def wait_gather(b):
        pltpu.make_async_copy(table_sh.at[idx_v.at[0]], rows[b], gsem[b]).wait()

    def wait_store(b):
        pltpu.make_async_copy(rows[b], out_hbm.at[pl.ds(0, CH)], ssem[b]).wait()

    is_tail_worker = wid < TAIL_WORKERS

    @pl.when(is_tail_worker)
    def _tail_start():
        pltpu.sync_copy(tail_hbm.at[pl.ds(wid * TAIL_PER_W, TAIL_PER_W)], tidx_v)
        pltpu.async_copy(table_sh.at[tidx_v], trows_v, psem)

    for b in range(LEAD):
        gather(b, b)

    def body(g, carry):
        for b in range(NBUF):
            c = g * NBUF + b
            bg = (b + LEAD) % NBUF
            if b + LEAD - NBUF >= 0:
                wait_store(bg)
            else:
                @pl.when(g > 0)
                def _():
                    wait_store(bg)

            @pl.when(c + LEAD < NCH)
            def _():
                gather(c + LEAD, bg)

            wait_gather(b)
            store(c, b)
        return carry

    lax.fori_loop(0, NGRP, body, 0)

    @pl.when(is_tail_worker)
    def _tail_finish():
        toff = pl.multiple_of(MAIN + wid * TAIL_PER_W, 8)
        pltpu.make_async_copy(table_sh.at[tidx_v], trows_v, psem).wait()
        pltpu.sync_copy(trows_v, out_hbm.at[pl.ds(toff, TAIL_PER_W)])

    for k in range(NCH - (NBUF - LEAD), NCH):
        wait_store(k % NBUF)


def kernel(node_attr, embedding_table):
    idx = node_attr.astype(jnp.int32)
    idx_main = idx[:MAIN].reshape(NW, NCH, CH)
    idx_tail = idx[MAIN:]
    return _emb_lookup(idx_main, idx_tail, embedding_table)
